# baseline (speedup 1.0000x reference)
import functools

import jax
import jax.numpy as jnp
from jax import lax
from jax.experimental import pallas as pl
from jax.experimental.pallas import tpu as pltpu
from jax.experimental.pallas import tpu_sc as plsc

_PELVIS = (0.5, 0.25, 0.25)
_NECK = (0.4, 0.3, 0.3)
_SHOULDER = (0.3, 0.2, 0.5)
_HIP = (0.6, 0.2, 0.2)

_L = 16
_NW = 32

_RAW_POS = {j: j for j in (0, 1, 2, 12, 13, 14, 16, 17)}
_RAW_ROWS = 24
_J14_ROWS = 8


def _plane_specs():
    specs = []
    for c in range(3):
        r = lambda jj: _RAW_POS[jj]
        specs += [
            (c, 8, [(_PELVIS[0], "r", r(0)), (_PELVIS[1], "r", r(1)), (_PELVIS[2], "r", r(2))]),
            (c, 1, [(_NECK[0], "r", r(12)), (_NECK[1], "r", r(13)), (_NECK[2], "r", r(14))]),
            (c, 5, [(_SHOULDER[0], "r", r(16)), (_SHOULDER[1], "r", r(12)), (_SHOULDER[2], "r", r(13))]),
            (c, 2, [(_SHOULDER[0], "r", r(17)), (_SHOULDER[1], "r", r(12)), (_SHOULDER[2], "r", r(14))]),
            (c, 12, [(_HIP[0], "r", r(1)), (_HIP[1], "r", r(0)), (_HIP[2], "j", 1)]),
            (c, 9, [(_HIP[0], "r", r(2)), (_HIP[1], "r", r(0)), (_HIP[2], "j", 4)]),
        ]
    return specs


def _sc_body(raw_hbm, j14_hbm, op_hbm, out_hbm, op_v, raw_v, j_v, sem_rj, sem_op):
    m = op_v.shape[2]
    wid = lax.axis_index("s") * 2 + lax.axis_index("c")
    wb = wid * m

    rj_waits = [
        pltpu.async_copy(
            raw_hbm.at[:, :, pl.ds(wb, m)], raw_v, sem_rj),
        pltpu.async_copy(
            j14_hbm.at[:, pl.ds(0, _J14_ROWS), pl.ds(wb, m)],
            j_v, sem_rj),
    ]
    op_wait = pltpu.async_copy(
        op_hbm.at[:, :, pl.ds(wb, m)], op_v, sem_op)
    for h in rj_waits:
        h.wait()

    specs = _plane_specs()

    def body(g, carry):
        k = g * _L
        for c, out_j, terms in specs:
            acc = None
            for w, arr, jj in terms:
                src = raw_v if arr == "r" else j_v
                v = w * src[c, jj, pl.ds(k, _L)]
                acc = v if acc is None else acc + v
            op_v[c, out_j, pl.ds(k, _L)] = acc
        return carry

    op_wait.wait()
    lax.fori_loop(0, m // _L, body, 0)
    pltpu.sync_copy(op_v, out_hbm.at[:, :, pl.ds(wb, m)])


def kernel(raw, j14, openpose):
    B = raw.shape[0]
    m = B // _NW

    mesh = plsc.VectorSubcoreMesh(core_axis_name="c", subcore_axis_name="s")
    f = functools.partial(
        pl.kernel,
        mesh=mesh,
        compiler_params=pltpu.CompilerParams(needs_layout_passes=False),
        out_type=jax.ShapeDtypeStruct((3, 25, B), jnp.float32),
        scratch_types=[
            pltpu.VMEM((3, 25, m), jnp.float32),
            pltpu.VMEM((3, _RAW_ROWS, m), jnp.float32),
            pltpu.VMEM((3, _J14_ROWS, m), jnp.float32),
            pltpu.SemaphoreType.DMA,
            pltpu.SemaphoreType.DMA,
        ],
    )(_sc_body)
    out = f(
        raw.transpose(2, 1, 0),
        j14.transpose(2, 1, 0),
        openpose.transpose(2, 1, 0),
    )
    return out.transpose(2, 1, 0)
